# Initial kernel scaffold; baseline (speedup 1.0000x reference)
import jax
import jax.numpy as jnp
from jax.experimental import pallas as pl


def kernel(feas_sim, scores, epoch):
    raise NotImplementedError("write your pallas kernel here")


# speedup vs baseline: 7.8293x; 7.8293x over previous
import functools

import jax
import jax.numpy as jnp
from jax import lax
from jax.experimental import pallas as pl
from jax.experimental.pallas import tpu as pltpu
from jax.experimental.pallas import tpu_sc as plsc

N = 100000
C = 100
D = 128
BN = 2048
NB = 49
NPAD = NB * BN
K1 = 362
K2 = 725
CAP = 1024
KP1 = 384
MAXI = 2**31 - 1
NEG = float("-inf")


def _tr_body(x_ref, o_ref):
    j = pl.program_id(0)
    xt = x_ref[...].T
    col = j * BN + lax.broadcasted_iota(jnp.int32, (C, BN), 1)
    o_ref[...] = jnp.where(col < N, xt, NEG)


def _transpose(scores):
    return pl.pallas_call(
        _tr_body,
        grid=(NB,),
        in_specs=[pl.BlockSpec((BN, C), lambda j: (j, 0))],
        out_specs=pl.BlockSpec((C, BN), lambda j: (0, j)),
        out_shape=jax.ShapeDtypeStruct((C, NPAD), jnp.float32),
    )(scores)


def _dis_body(s_ref, f_ref, o_ref):
    j = pl.program_id(0)
    dis = lax.dot_general(s_ref[...], f_ref[...], (((1,), (1,)), ((), ())),
                          preferred_element_type=jnp.float32)
    col = j * BN + lax.broadcasted_iota(jnp.int32, (C, BN), 1)
    o_ref[...] = jnp.where(col < N, dis, NEG)


def _dis(sums, feas):
    return pl.pallas_call(
        _dis_body,
        grid=(NB,),
        in_specs=[pl.BlockSpec((C, D), lambda j: (0, 0)),
                  pl.BlockSpec((BN, D), lambda j: (j, 0))],
        out_specs=pl.BlockSpec((C, BN), lambda j: (0, j)),
        out_shape=jax.ShapeDtypeStruct((C, NPAD), jnp.float32),
    )(sums, feas)


def _sort_body(v_ref, i_ref, oi_ref):
    val = v_ref[...]
    idx = i_ref[...]
    pos = lax.broadcasted_iota(jnp.int32, (C, CAP), 1)
    s = 2
    while s <= CAP:
        d = s // 2
        while d >= 1:
            low = (pos & d) == 0
            pv = jnp.where(low, jnp.roll(val, -d, axis=1), jnp.roll(val, d, axis=1))
            pi = jnp.where(low, jnp.roll(idx, -d, axis=1), jnp.roll(idx, d, axis=1))
            before = (val > pv) | ((val == pv) & (idx < pi))
            dirup = (pos & s) == 0
            take_own = ~(before ^ ~(dirup ^ low))
            val = jnp.where(take_own, val, pv)
            idx = jnp.where(take_own, idx, pi)
            d //= 2
        s *= 2
    oi_ref[...] = idx


def _sort_cands(cval, cidx):
    return pl.pallas_call(
        _sort_body,
        in_specs=[pl.BlockSpec((C, CAP), lambda: (0, 0)),
                  pl.BlockSpec((C, CAP), lambda: (0, 0))],
        out_specs=pl.BlockSpec((C, CAP), lambda: (0, 0)),
        out_shape=jax.ShapeDtypeStruct((C, CAP), jnp.int32),
    )(cval, cidx)


_NCHUNK = NPAD // 16


@functools.cache
def _mesh():
    return plsc.VectorSubcoreMesh(core_axis_name="c", subcore_axis_name="s",
                                  num_cores=2)


def _key_u32(v):
    b = lax.bitcast_convert_type(v, jnp.int32)
    bu = lax.bitcast_convert_type(b, jnp.uint32)
    return jnp.where(b < 0, ~bu, bu | jnp.uint32(0x80000000))


def _hist_pass(row_v, hist_v, lane, shift, pmask, pval):
    def zero(i, _):
        hist_v[pl.ds(i * 16, 16)] = jnp.zeros((16,), jnp.int32)
        return 0
    lax.fori_loop(0, 256, zero, 0)

    def body(i, _):
        key = _key_u32(row_v[pl.ds(i * 16, 16)])
        dig = lax.convert_element_type(
            lax.shift_right_logical(key, jnp.uint32(shift)) & jnp.uint32(0xFF),
            jnp.int32)
        ok = (key & pmask) == pval
        plsc.addupdate_scatter(hist_v, [dig * 16 + lane],
                               jnp.ones((16,), jnp.int32), mask=ok)
        return 0
    lax.fori_loop(0, _NCHUNK, body, 0)


def _hist_scan(hist_v, k, base):
    def body(j, carry):
        tot, dsel, above = carry
        d = 255 - j
        cnt = jnp.sum(hist_v[pl.ds(d * 16, 16)])
        newtot = tot + cnt
        first = (newtot + base >= k) & (tot + base < k)
        dsel = jnp.where(first, d, dsel)
        above = jnp.where(first, tot + base, above)
        return newtot, dsel, above
    _, dsel, above = lax.fori_loop(
        0, 256, body, (jnp.int32(0), jnp.int32(0), jnp.int32(0)))
    return dsel, above


def _select_body(k, x_hbm, ov_hbm, oi_hbm, row_v, hist_v, cval_v, cidx_v, sem):
    wid = lax.axis_index("s") * 2 + lax.axis_index("c")
    lane = lax.iota(jnp.int32, 16)
    for t in range(4):
        r = wid + 32 * t
        @pl.when(r < C)
        def _():
            pltpu.sync_copy(x_hbm.at[r], row_v)
            _hist_pass(row_v, hist_v, lane, 24, jnp.uint32(0), jnp.uint32(0))
            d1, above1 = _hist_scan(hist_v, k, jnp.int32(0))
            p1 = lax.convert_element_type(d1, jnp.uint32) << jnp.uint32(24)
            _hist_pass(row_v, hist_v, lane, 16, jnp.uint32(0xFF000000), p1)
            d2, above2 = _hist_scan(hist_v, k, above1)
            p2 = p1 | (lax.convert_element_type(d2, jnp.uint32) << jnp.uint32(16))
            _hist_pass(row_v, hist_v, lane, 8, jnp.uint32(0xFFFF0000), p2)
            d3, _ = _hist_scan(hist_v, k, above2)
            thr = p2 | (lax.convert_element_type(d3, jnp.uint32) << jnp.uint32(8))

            def fill(i, _):
                cval_v[pl.ds(i * 16, 16)] = jnp.full((16,), NEG, jnp.float32)
                cidx_v[pl.ds(i * 16, 16)] = jnp.full((16,), MAXI, jnp.int32)
                return 0
            lax.fori_loop(0, CAP // 16, fill, 0)

            def compact(i, ptr):
                v = row_v[pl.ds(i * 16, 16)]
                m = _key_u32(v) >= thr
                inc = plsc.cumsum(jnp.where(m, 1, 0))
                post = jnp.broadcast_to(ptr, (16,)) + inc
                ok = m & (post <= CAP)
                plsc.store_scatter(cval_v, [post - 1], v, mask=ok)
                plsc.store_scatter(cidx_v, [post - 1], i * 16 + lane, mask=ok)
                return ptr + jnp.sum(jnp.where(ok, 1, 0))
            lax.fori_loop(0, _NCHUNK, compact, jnp.int32(0))
            pltpu.sync_copy(cval_v, ov_hbm.at[r])
            pltpu.sync_copy(cidx_v, oi_hbm.at[r])


def _select(x, k):
    kern = pl.kernel(
        functools.partial(_select_body, k),
        out_type=(jax.ShapeDtypeStruct((C, CAP), jnp.float32),
                  jax.ShapeDtypeStruct((C, CAP), jnp.int32)),
        mesh=_mesh(),
        scratch_types=[
            pltpu.VMEM((NPAD,), jnp.float32),
            pltpu.VMEM((256 * 16,), jnp.int32),
            pltpu.VMEM((CAP,), jnp.float32),
            pltpu.VMEM((CAP,), jnp.int32),
            pltpu.SemaphoreType.DMA,
        ],
        compiler_params=pltpu.CompilerParams(needs_layout_passes=False),
    )
    return kern(x)


def _gather_body(feas_hbm, idx_hbm, out_hbm, idx_v, rows_v, acc_v, sem):
    wid = lax.axis_index("s") * 2 + lax.axis_index("c")
    lane = lax.iota(jnp.int32, 16)
    for t in range(4):
        r = wid + 32 * t
        @pl.when(r < C)
        def _():
            pltpu.sync_copy(idx_hbm.at[r, pl.ds(0, KP1)], idx_v)
            v22 = idx_v[pl.ds(352, 16)]
            idx_v[pl.ds(352, 16)] = jnp.where(352 + lane < K1, v22, 0)
            idx_v[pl.ds(368, 16)] = jnp.zeros((16,), jnp.int32)
            pltpu.async_copy(feas_hbm.at[idx_v], rows_v, sem).wait()

            def body(rr, carry):
                return tuple(
                    carry[j] + rows_v[rr, pl.ds(j * 16, 16)] for j in range(8))
            acc = lax.fori_loop(
                0, K1, body, tuple(jnp.zeros((16,), jnp.float32) for _ in range(8)))
            for j in range(8):
                acc_v[pl.ds(j * 16, 16)] = acc[j]
            pltpu.sync_copy(acc_v, out_hbm.at[r])


def _gather_sums(feas, idxk):
    kern = pl.kernel(
        _gather_body,
        out_type=jax.ShapeDtypeStruct((C, D), jnp.float32),
        mesh=_mesh(),
        scratch_types=[
            pltpu.VMEM((KP1,), jnp.int32),
            pltpu.VMEM((KP1, D), jnp.float32),
            pltpu.VMEM((D,), jnp.float32),
            pltpu.SemaphoreType.DMA,
        ],
        compiler_params=pltpu.CompilerParams(needs_layout_passes=False),
    )
    return kern(feas, idxk)


def kernel(feas_sim, scores, epoch):
    scoresT = _transpose(scores)
    cval1, cidx1 = _select(scoresT, K1)
    sidx1 = _sort_cands(cval1, cidx1)
    idx_max_k = sidx1[:, :K1].T
    sums = _gather_sums(feas_sim, sidx1[:, :KP1])
    centers = sums * (1.0 / 362.0)
    centers = centers / jnp.linalg.norm(centers, axis=1, keepdims=True)
    dis = _dis(centers, feas_sim)
    cval2, cidx2 = _select(dis, K2)
    sidx2 = _sort_cands(cval2, cidx2)
    idx_select = sidx2[:, :K2].reshape(-1)
    idx_select = idx_select + (jnp.asarray(epoch) * 0).astype(jnp.int32)
    labels_select = jnp.repeat(jnp.arange(C, dtype=jnp.int32), K2)
    return (idx_select, labels_select, idx_max_k)


# speedup vs baseline: 13.1344x; 1.6776x over previous
import functools

import jax
import jax.numpy as jnp
from jax import lax
from jax.experimental import pallas as pl
from jax.experimental.pallas import tpu as pltpu
from jax.experimental.pallas import tpu_sc as plsc

N = 100000
C = 100
D = 128
BN = 2048
NB = 49
NPAD = NB * BN
K1 = 362
K2 = 725
CAP = 1024
KP1 = 384
MAXI = 2**31 - 1
NEG = float("-inf")


def _tr_body(x_ref, o_ref):
    j = pl.program_id(0)
    xt = x_ref[...].T
    col = j * BN + lax.broadcasted_iota(jnp.int32, (C, BN), 1)
    o_ref[...] = jnp.where(col < N, xt, NEG)


def _transpose(scores):
    return pl.pallas_call(
        _tr_body,
        grid=(NB,),
        in_specs=[pl.BlockSpec((BN, C), lambda j: (j, 0))],
        out_specs=pl.BlockSpec((C, BN), lambda j: (0, j)),
        out_shape=jax.ShapeDtypeStruct((C, NPAD), jnp.float32),
    )(scores)


def _dis_body(s_ref, f_ref, o_ref):
    j = pl.program_id(0)
    dis = lax.dot_general(s_ref[...], f_ref[...], (((1,), (1,)), ((), ())),
                          preferred_element_type=jnp.float32)
    col = j * BN + lax.broadcasted_iota(jnp.int32, (C, BN), 1)
    o_ref[...] = jnp.where(col < N, dis, NEG)


def _dis(sums, feas):
    return pl.pallas_call(
        _dis_body,
        grid=(NB,),
        in_specs=[pl.BlockSpec((C, D), lambda j: (0, 0)),
                  pl.BlockSpec((BN, D), lambda j: (j, 0))],
        out_specs=pl.BlockSpec((C, BN), lambda j: (0, j)),
        out_shape=jax.ShapeDtypeStruct((C, NPAD), jnp.float32),
    )(sums, feas)


def _sort_body(v_ref, i_ref, oi_ref):
    val = v_ref[...]
    idx = i_ref[...]
    pos = lax.broadcasted_iota(jnp.int32, (C, CAP), 1)
    s = 2
    while s <= CAP:
        d = s // 2
        while d >= 1:
            low = (pos & d) == 0
            pv = jnp.where(low, jnp.roll(val, -d, axis=1), jnp.roll(val, d, axis=1))
            pi = jnp.where(low, jnp.roll(idx, -d, axis=1), jnp.roll(idx, d, axis=1))
            before = (val > pv) | ((val == pv) & (idx < pi))
            dirup = (pos & s) == 0
            take_own = ~(before ^ ~(dirup ^ low))
            val = jnp.where(take_own, val, pv)
            idx = jnp.where(take_own, idx, pi)
            d //= 2
        s *= 2
    oi_ref[...] = idx


def _sort_cands(cval, cidx):
    return pl.pallas_call(
        _sort_body,
        in_specs=[pl.BlockSpec((C, CAP), lambda: (0, 0)),
                  pl.BlockSpec((C, CAP), lambda: (0, 0))],
        out_specs=pl.BlockSpec((C, CAP), lambda: (0, 0)),
        out_shape=jax.ShapeDtypeStruct((C, CAP), jnp.int32),
    )(cval, cidx)


_NCHUNK = NPAD // 16


@functools.cache
def _mesh():
    return plsc.VectorSubcoreMesh(core_axis_name="c", subcore_axis_name="s",
                                  num_cores=2)


def _key_u32(v):
    b = lax.bitcast_convert_type(v, jnp.int32)
    bu = lax.bitcast_convert_type(b, jnp.uint32)
    return jnp.where(b < 0, ~bu, bu | jnp.uint32(0x80000000))


_UNR = 4
_NGROUP = _NCHUNK // _UNR


def _hist_pass(row_v, hist_v, lane, shift, pmask, pval):
    ones = jnp.ones((16,), jnp.int32)

    def zero(i, _):
        for u in range(8):
            hist_v[pl.ds(i * 128 + u * 16, 16)] = jnp.zeros((16,), jnp.int32)
        return 0
    lax.fori_loop(0, 32, zero, 0)

    def body(i, _):
        base = i * (16 * _UNR)
        for u in range(_UNR):
            key = _key_u32(row_v[pl.ds(base + u * 16, 16)])
            dig = lax.convert_element_type(
                lax.shift_right_logical(key, jnp.uint32(shift))
                & jnp.uint32(0xFF), jnp.int32)
            ok = (key & pmask) == pval
            plsc.addupdate_scatter(hist_v, [dig * 16 + lane], ones, mask=ok)
        return 0
    lax.fori_loop(0, _NGROUP, body, 0)


def _hist_scan(hist_v, k, base):
    def body(j, carry):
        tot, dsel, above, occ = carry
        d = 255 - j
        cnt = jnp.sum(hist_v[pl.ds(d * 16, 16)])
        newtot = tot + cnt
        first = (newtot + base >= k) & (tot + base < k)
        dsel = jnp.where(first, d, dsel)
        above = jnp.where(first, tot + base, above)
        occ = jnp.where(first, cnt, occ)
        return newtot, dsel, above, occ
    _, dsel, above, occ = lax.fori_loop(
        0, 256, body,
        (jnp.int32(0), jnp.int32(0), jnp.int32(0), jnp.int32(0)))
    return dsel, above, occ


def _select_body(k, x_hbm, ov_hbm, oi_hbm, row_v, hist_v, cval_v, cidx_v,
                 thr_s, sem):
    wid = lax.axis_index("s") * 2 + lax.axis_index("c")
    lane = lax.iota(jnp.int32, 16)
    for t in range(4):
        r = wid + 32 * t
        @pl.when(r < C)
        def _():
            pltpu.sync_copy(x_hbm.at[r], row_v)
            _hist_pass(row_v, hist_v, lane, 24, jnp.uint32(0), jnp.uint32(0))
            d1, above1, _ = _hist_scan(hist_v, k, jnp.int32(0))
            p1 = lax.convert_element_type(d1, jnp.uint32) << jnp.uint32(24)
            _hist_pass(row_v, hist_v, lane, 16, jnp.uint32(0xFF000000), p1)
            d2, above2, occ2 = _hist_scan(hist_v, k, above1)
            p2 = p1 | (lax.convert_element_type(d2, jnp.uint32) << jnp.uint32(16))
            thr_s[0] = p2
            @pl.when(above2 + occ2 > CAP)
            def _():
                _hist_pass(row_v, hist_v, lane, 8, jnp.uint32(0xFFFF0000), p2)
                d3, _, _ = _hist_scan(hist_v, k, above2)
                thr_s[0] = p2 | (lax.convert_element_type(d3, jnp.uint32)
                                 << jnp.uint32(8))
            thr = thr_s[0]

            def fill(i, _):
                for u in range(4):
                    cval_v[pl.ds(i * 64 + u * 16, 16)] = jnp.full(
                        (16,), NEG, jnp.float32)
                    cidx_v[pl.ds(i * 64 + u * 16, 16)] = jnp.full(
                        (16,), MAXI, jnp.int32)
                return 0
            lax.fori_loop(0, CAP // 64, fill, 0)

            def compact(i, ptr):
                base = i * (16 * _UNR)
                incs, oks, vs = [], [], []
                for u in range(_UNR):
                    v = row_v[pl.ds(base + u * 16, 16)]
                    m = _key_u32(v) >= thr
                    mi = jnp.where(m, 1, 0)
                    vs.append(v)
                    oks.append(m)
                    incs.append((plsc.cumsum(mi), jnp.sum(mi)))
                tot = ptr
                for u in range(_UNR):
                    inc, cnt = incs[u]
                    post = jnp.broadcast_to(tot, (16,)) + inc
                    ok = oks[u] & (post <= CAP)
                    plsc.store_scatter(cval_v, [post - 1], vs[u], mask=ok)
                    plsc.store_scatter(cidx_v, [post - 1],
                                       base + u * 16 + lane, mask=ok)
                    tot = tot + cnt
                return tot
            lax.fori_loop(0, _NGROUP, compact, jnp.int32(0))
            pltpu.sync_copy(cval_v, ov_hbm.at[r])
            pltpu.sync_copy(cidx_v, oi_hbm.at[r])


def _select(x, k):
    kern = pl.kernel(
        functools.partial(_select_body, k),
        out_type=(jax.ShapeDtypeStruct((C, CAP), jnp.float32),
                  jax.ShapeDtypeStruct((C, CAP), jnp.int32)),
        mesh=_mesh(),
        scratch_types=[
            pltpu.VMEM((NPAD,), jnp.float32),
            pltpu.VMEM((256 * 16,), jnp.int32),
            pltpu.VMEM((CAP,), jnp.float32),
            pltpu.VMEM((CAP,), jnp.int32),
            pltpu.SMEM((1,), jnp.uint32),
            pltpu.SemaphoreType.DMA,
        ],
        compiler_params=pltpu.CompilerParams(needs_layout_passes=False),
    )
    return kern(x)


def _gather_body(feas_hbm, idx_hbm, out_hbm, idx_v, rows_v, acc_v, sem):
    wid = lax.axis_index("s") * 2 + lax.axis_index("c")
    lane = lax.iota(jnp.int32, 16)
    for t in range(4):
        r = wid + 32 * t
        @pl.when(r < C)
        def _():
            pltpu.sync_copy(idx_hbm.at[r, pl.ds(0, KP1)], idx_v)
            v22 = idx_v[pl.ds(352, 16)]
            idx_v[pl.ds(352, 16)] = jnp.where(352 + lane < K1, v22, 0)
            idx_v[pl.ds(368, 16)] = jnp.zeros((16,), jnp.int32)
            pltpu.async_copy(feas_hbm.at[idx_v], rows_v, sem).wait()

            def body(rr, carry):
                return tuple(
                    carry[j] + rows_v[rr, pl.ds(j * 16, 16)] for j in range(8))
            acc = lax.fori_loop(
                0, K1, body, tuple(jnp.zeros((16,), jnp.float32) for _ in range(8)))
            for j in range(8):
                acc_v[pl.ds(j * 16, 16)] = acc[j]
            pltpu.sync_copy(acc_v, out_hbm.at[r])


def _gather_sums(feas, idxk):
    kern = pl.kernel(
        _gather_body,
        out_type=jax.ShapeDtypeStruct((C, D), jnp.float32),
        mesh=_mesh(),
        scratch_types=[
            pltpu.VMEM((KP1,), jnp.int32),
            pltpu.VMEM((KP1, D), jnp.float32),
            pltpu.VMEM((D,), jnp.float32),
            pltpu.SemaphoreType.DMA,
        ],
        compiler_params=pltpu.CompilerParams(needs_layout_passes=False),
    )
    return kern(feas, idxk)


def kernel(feas_sim, scores, epoch):
    scoresT = _transpose(scores)
    cval1, cidx1 = _select(scoresT, K1)
    sidx1 = _sort_cands(cval1, cidx1)
    idx_max_k = sidx1[:, :K1].T
    sums = _gather_sums(feas_sim, sidx1[:, :KP1])
    centers = sums * (1.0 / 362.0)
    centers = centers / jnp.linalg.norm(centers, axis=1, keepdims=True)
    dis = _dis(centers, feas_sim)
    cval2, cidx2 = _select(dis, K2)
    sidx2 = _sort_cands(cval2, cidx2)
    idx_select = sidx2[:, :K2].reshape(-1)
    idx_select = idx_select + (jnp.asarray(epoch) * 0).astype(jnp.int32)
    labels_select = jnp.repeat(jnp.arange(C, dtype=jnp.int32), K2)
    return (idx_select, labels_select, idx_max_k)


# speedup vs baseline: 14.9249x; 1.1363x over previous
import functools

import jax
import jax.numpy as jnp
from jax import lax
from jax.experimental import pallas as pl
from jax.experimental.pallas import tpu as pltpu
from jax.experimental.pallas import tpu_sc as plsc

N = 100000
C = 100
D = 128
BN = 2048
NB = 49
NPAD = NB * BN
K1 = 362
K2 = 725
CAP = 1024
KP1 = 384
MAXI = 2**31 - 1
NEG = float("-inf")


def _tr_body(x_ref, o_ref):
    j = pl.program_id(0)
    xt = x_ref[...].T
    col = j * BN + lax.broadcasted_iota(jnp.int32, (C, BN), 1)
    o_ref[...] = jnp.where(col < N, xt, NEG)


def _transpose(scores):
    return pl.pallas_call(
        _tr_body,
        grid=(NB,),
        in_specs=[pl.BlockSpec((BN, C), lambda j: (j, 0))],
        out_specs=pl.BlockSpec((C, BN), lambda j: (0, j)),
        out_shape=jax.ShapeDtypeStruct((C, NPAD), jnp.float32),
    )(scores)


def _dis_body(s_ref, f_ref, o_ref):
    j = pl.program_id(0)
    dis = lax.dot_general(s_ref[...], f_ref[...], (((1,), (1,)), ((), ())),
                          preferred_element_type=jnp.float32)
    col = j * BN + lax.broadcasted_iota(jnp.int32, (C, BN), 1)
    o_ref[...] = jnp.where(col < N, dis, NEG)


def _dis(sums, feas):
    return pl.pallas_call(
        _dis_body,
        grid=(NB,),
        in_specs=[pl.BlockSpec((C, D), lambda j: (0, 0)),
                  pl.BlockSpec((BN, D), lambda j: (j, 0))],
        out_specs=pl.BlockSpec((C, BN), lambda j: (0, j)),
        out_shape=jax.ShapeDtypeStruct((C, NPAD), jnp.float32),
    )(sums, feas)


def _sort_body(v_ref, i_ref, oi_ref):
    val = v_ref[...]
    idx = i_ref[...]
    pos = lax.broadcasted_iota(jnp.int32, (C, CAP), 1)
    s = 2
    while s <= CAP:
        d = s // 2
        while d >= 1:
            low = (pos & d) == 0
            pv = jnp.where(low, jnp.roll(val, -d, axis=1), jnp.roll(val, d, axis=1))
            pi = jnp.where(low, jnp.roll(idx, -d, axis=1), jnp.roll(idx, d, axis=1))
            before = (val > pv) | ((val == pv) & (idx < pi))
            dirup = (pos & s) == 0
            take_own = ~(before ^ ~(dirup ^ low))
            val = jnp.where(take_own, val, pv)
            idx = jnp.where(take_own, idx, pi)
            d //= 2
        s *= 2
    oi_ref[...] = idx


def _sort_cands(cval, cidx):
    return pl.pallas_call(
        _sort_body,
        in_specs=[pl.BlockSpec((C, CAP), lambda: (0, 0)),
                  pl.BlockSpec((C, CAP), lambda: (0, 0))],
        out_specs=pl.BlockSpec((C, CAP), lambda: (0, 0)),
        out_shape=jax.ShapeDtypeStruct((C, CAP), jnp.int32),
    )(cval, cidx)


_NCHUNK = NPAD // 16


@functools.cache
def _mesh():
    return plsc.VectorSubcoreMesh(core_axis_name="c", subcore_axis_name="s",
                                  num_cores=2)


def _key_u32(v):
    b = lax.bitcast_convert_type(v, jnp.int32)
    bu = lax.bitcast_convert_type(b, jnp.uint32)
    return jnp.where(b < 0, ~bu, bu | jnp.uint32(0x80000000))


_UNR = 8
_NGROUP = _NCHUNK // _UNR


def _hist_pass(row_v, hist_v, lane, shift, pmask, pval):
    ones = jnp.ones((16,), jnp.int32)

    def zero(i, _):
        for u in range(8):
            hist_v[pl.ds(i * 128 + u * 16, 16)] = jnp.zeros((16,), jnp.int32)
        return 0
    lax.fori_loop(0, 32, zero, 0)

    def body(i, _):
        base = i * (16 * _UNR)
        for u in range(_UNR):
            key = _key_u32(row_v[pl.ds(base + u * 16, 16)])
            dig = lax.convert_element_type(
                lax.shift_right_logical(key, jnp.uint32(shift))
                & jnp.uint32(0xFF), jnp.int32)
            ok = (key & pmask) == pval
            plsc.addupdate_scatter(hist_v, [dig * 16 + lane], ones, mask=ok)
        return 0
    lax.fori_loop(0, _NGROUP, body, 0)


def _hist_scan(hist_v, k, base):
    def body(j, carry):
        tot, dsel, above, occ = carry
        d = 255 - j
        cnt = jnp.sum(hist_v[pl.ds(d * 16, 16)])
        newtot = tot + cnt
        first = (newtot + base >= k) & (tot + base < k)
        dsel = jnp.where(first, d, dsel)
        above = jnp.where(first, tot + base, above)
        occ = jnp.where(first, cnt, occ)
        return newtot, dsel, above, occ
    _, dsel, above, occ = lax.fori_loop(
        0, 256, body,
        (jnp.int32(0), jnp.int32(0), jnp.int32(0), jnp.int32(0)))
    return dsel, above, occ


def _select_body(k, x_hbm, ov_hbm, oi_hbm, row_v, hist_v, cval_v, cidx_v,
                 thr_s, sem):
    wid = lax.axis_index("s") * 2 + lax.axis_index("c")
    lane = lax.iota(jnp.int32, 16)
    for t in range(4):
        r = wid + 32 * t
        @pl.when(r < C)
        def _():
            pltpu.sync_copy(x_hbm.at[r], row_v)
            _hist_pass(row_v, hist_v, lane, 24, jnp.uint32(0), jnp.uint32(0))
            d1, above1, _ = _hist_scan(hist_v, k, jnp.int32(0))
            p1 = lax.convert_element_type(d1, jnp.uint32) << jnp.uint32(24)
            _hist_pass(row_v, hist_v, lane, 16, jnp.uint32(0xFF000000), p1)
            d2, above2, occ2 = _hist_scan(hist_v, k, above1)
            p2 = p1 | (lax.convert_element_type(d2, jnp.uint32) << jnp.uint32(16))
            thr_s[0] = p2
            @pl.when(above2 + occ2 > CAP)
            def _():
                _hist_pass(row_v, hist_v, lane, 8, jnp.uint32(0xFFFF0000), p2)
                d3, _, _ = _hist_scan(hist_v, k, above2)
                thr_s[0] = p2 | (lax.convert_element_type(d3, jnp.uint32)
                                 << jnp.uint32(8))
            thr = thr_s[0]

            def fill(i, _):
                for u in range(4):
                    cval_v[pl.ds(i * 64 + u * 16, 16)] = jnp.full(
                        (16,), NEG, jnp.float32)
                    cidx_v[pl.ds(i * 64 + u * 16, 16)] = jnp.full(
                        (16,), MAXI, jnp.int32)
                return 0
            lax.fori_loop(0, CAP // 64, fill, 0)

            def compact(i, ptrv):
                base = i * (16 * _UNR)
                datas = []
                for u in range(_UNR):
                    v = row_v[pl.ds(base + u * 16, 16)]
                    m = _key_u32(v) >= thr
                    inc = plsc.cumsum(jnp.where(m, 1, 0))
                    pc = plsc.all_reduce_population_count(m)
                    datas.append((v, m, inc, pc))
                for u in range(_UNR):
                    v, m, inc, pc = datas[u]
                    post = ptrv + inc
                    ok = m & (post <= CAP)
                    plsc.store_scatter(cval_v, [post - 1], v, mask=ok)
                    plsc.store_scatter(cidx_v, [post - 1],
                                       base + u * 16 + lane, mask=ok)
                    ptrv = ptrv + pc
                return ptrv
            lax.fori_loop(0, _NGROUP, compact, jnp.zeros((16,), jnp.int32))
            pltpu.sync_copy(cval_v, ov_hbm.at[r])
            pltpu.sync_copy(cidx_v, oi_hbm.at[r])


def _select(x, k):
    kern = pl.kernel(
        functools.partial(_select_body, k),
        out_type=(jax.ShapeDtypeStruct((C, CAP), jnp.float32),
                  jax.ShapeDtypeStruct((C, CAP), jnp.int32)),
        mesh=_mesh(),
        scratch_types=[
            pltpu.VMEM((NPAD,), jnp.float32),
            pltpu.VMEM((256 * 16,), jnp.int32),
            pltpu.VMEM((CAP,), jnp.float32),
            pltpu.VMEM((CAP,), jnp.int32),
            pltpu.SMEM((1,), jnp.uint32),
            pltpu.SemaphoreType.DMA,
        ],
        compiler_params=pltpu.CompilerParams(needs_layout_passes=False),
    )
    return kern(x)


def _gather_body(feas_hbm, idx_hbm, out_hbm, idx_v, rows_v, acc_v, sem):
    wid = lax.axis_index("s") * 2 + lax.axis_index("c")
    lane = lax.iota(jnp.int32, 16)
    for t in range(4):
        r = wid + 32 * t
        @pl.when(r < C)
        def _():
            pltpu.sync_copy(idx_hbm.at[r, pl.ds(0, KP1)], idx_v)
            v22 = idx_v[pl.ds(352, 16)]
            idx_v[pl.ds(352, 16)] = jnp.where(352 + lane < K1, v22, 0)
            idx_v[pl.ds(368, 16)] = jnp.zeros((16,), jnp.int32)
            pltpu.async_copy(feas_hbm.at[idx_v], rows_v, sem).wait()

            def body(rr, carry):
                return tuple(
                    carry[j] + rows_v[rr, pl.ds(j * 16, 16)] for j in range(8))
            acc = lax.fori_loop(
                0, K1, body, tuple(jnp.zeros((16,), jnp.float32) for _ in range(8)))
            for j in range(8):
                acc_v[pl.ds(j * 16, 16)] = acc[j]
            pltpu.sync_copy(acc_v, out_hbm.at[r])


def _gather_sums(feas, idxk):
    kern = pl.kernel(
        _gather_body,
        out_type=jax.ShapeDtypeStruct((C, D), jnp.float32),
        mesh=_mesh(),
        scratch_types=[
            pltpu.VMEM((KP1,), jnp.int32),
            pltpu.VMEM((KP1, D), jnp.float32),
            pltpu.VMEM((D,), jnp.float32),
            pltpu.SemaphoreType.DMA,
        ],
        compiler_params=pltpu.CompilerParams(needs_layout_passes=False),
    )
    return kern(feas, idxk)


def kernel(feas_sim, scores, epoch):
    scoresT = _transpose(scores)
    cval1, cidx1 = _select(scoresT, K1)
    sidx1 = _sort_cands(cval1, cidx1)
    idx_max_k = sidx1[:, :K1].T
    sums = _gather_sums(feas_sim, sidx1[:, :KP1])
    centers = sums * (1.0 / 362.0)
    centers = centers / jnp.linalg.norm(centers, axis=1, keepdims=True)
    dis = _dis(centers, feas_sim)
    cval2, cidx2 = _select(dis, K2)
    sidx2 = _sort_cands(cval2, cidx2)
    idx_select = sidx2[:, :K2].reshape(-1)
    idx_select = idx_select + (jnp.asarray(epoch) * 0).astype(jnp.int32)
    labels_select = jnp.repeat(jnp.arange(C, dtype=jnp.int32), K2)
    return (idx_select, labels_select, idx_max_k)


# speedup vs baseline: 29.8898x; 2.0027x over previous
import functools

import jax
import jax.numpy as jnp
from jax import lax
from jax.experimental import pallas as pl
from jax.experimental.pallas import tpu as pltpu
from jax.experimental.pallas import tpu_sc as plsc

N = 100000
C = 100
D = 128
BN = 2048
NB = 49
NPAD = NB * BN
K1 = 362
K2 = 725
CAP = 1024
KP1 = 384
MAXI = 2**31 - 1
NEG = float("-inf")


def _tr_body(x_ref, o_ref):
    j = pl.program_id(0)
    xt = x_ref[...].T
    col = j * BN + lax.broadcasted_iota(jnp.int32, (C, BN), 1)
    o_ref[...] = jnp.where(col < N, xt, NEG)


def _transpose(scores):
    return pl.pallas_call(
        _tr_body,
        grid=(NB,),
        in_specs=[pl.BlockSpec((BN, C), lambda j: (j, 0))],
        out_specs=pl.BlockSpec((C, BN), lambda j: (0, j)),
        out_shape=jax.ShapeDtypeStruct((C, NPAD), jnp.float32),
    )(scores)


def _dis_body(s_ref, f_ref, o_ref):
    j = pl.program_id(0)
    dis = lax.dot_general(s_ref[...], f_ref[...], (((1,), (1,)), ((), ())),
                          preferred_element_type=jnp.float32)
    col = j * BN + lax.broadcasted_iota(jnp.int32, (C, BN), 1)
    o_ref[...] = jnp.where(col < N, dis, NEG)


def _dis(sums, feas):
    return pl.pallas_call(
        _dis_body,
        grid=(NB,),
        in_specs=[pl.BlockSpec((C, D), lambda j: (0, 0)),
                  pl.BlockSpec((BN, D), lambda j: (j, 0))],
        out_specs=pl.BlockSpec((C, BN), lambda j: (0, j)),
        out_shape=jax.ShapeDtypeStruct((C, NPAD), jnp.float32),
    )(sums, feas)


def _sort_body(v_ref, i_ref, oi_ref):
    val = v_ref[...]
    idx = i_ref[...]
    pos = lax.broadcasted_iota(jnp.int32, (C, CAP), 1)
    s = 2
    while s <= CAP:
        d = s // 2
        while d >= 1:
            low = (pos & d) == 0
            pv = jnp.where(low, jnp.roll(val, -d, axis=1), jnp.roll(val, d, axis=1))
            pi = jnp.where(low, jnp.roll(idx, -d, axis=1), jnp.roll(idx, d, axis=1))
            before = (val > pv) | ((val == pv) & (idx < pi))
            dirup = (pos & s) == 0
            take_own = ~(before ^ ~(dirup ^ low))
            val = jnp.where(take_own, val, pv)
            idx = jnp.where(take_own, idx, pi)
            d //= 2
        s *= 2
    oi_ref[...] = idx


def _sort_cands(cval, cidx):
    return pl.pallas_call(
        _sort_body,
        in_specs=[pl.BlockSpec((C, CAP), lambda: (0, 0)),
                  pl.BlockSpec((C, CAP), lambda: (0, 0))],
        out_specs=pl.BlockSpec((C, CAP), lambda: (0, 0)),
        out_shape=jax.ShapeDtypeStruct((C, CAP), jnp.int32),
    )(cval, cidx)


_NCHUNK = NPAD // 16


@functools.cache
def _mesh():
    return plsc.VectorSubcoreMesh(core_axis_name="c", subcore_axis_name="s",
                                  num_cores=2)


def _key_u32(v):
    b = lax.bitcast_convert_type(v, jnp.int32)
    bu = lax.bitcast_convert_type(b, jnp.uint32)
    return jnp.where(b < 0, ~bu, bu | jnp.uint32(0x80000000))


_UNR = 8
_NGROUP = _NCHUNK // _UNR


def _hist_pass(row_v, hist_v, lane, shift, pmask, pval):
    ones = jnp.ones((16,), jnp.int32)

    def zero(i, _):
        for u in range(8):
            hist_v[pl.ds(i * 128 + u * 16, 16)] = jnp.zeros((16,), jnp.int32)
        return 0
    lax.fori_loop(0, 32, zero, 0)

    @plsc.parallel_loop(0, _NGROUP)
    def _(i):
        base = i * (16 * _UNR)
        for u in range(_UNR):
            key = _key_u32(row_v[pl.ds(base + u * 16, 16)])
            dig = lax.convert_element_type(
                lax.shift_right_logical(key, jnp.uint32(shift))
                & jnp.uint32(0xFF), jnp.int32)
            ok = (key & pmask) == pval
            plsc.addupdate_scatter(hist_v, [dig * 16 + lane], ones, mask=ok)


def _hist_scan(hist_v, k, base):
    def body(j, carry):
        tot, dsel, above, occ = carry
        d = 255 - j
        cnt = jnp.sum(hist_v[pl.ds(d * 16, 16)])
        newtot = tot + cnt
        first = (newtot + base >= k) & (tot + base < k)
        dsel = jnp.where(first, d, dsel)
        above = jnp.where(first, tot + base, above)
        occ = jnp.where(first, cnt, occ)
        return newtot, dsel, above, occ
    _, dsel, above, occ = lax.fori_loop(
        0, 256, body,
        (jnp.int32(0), jnp.int32(0), jnp.int32(0), jnp.int32(0)))
    return dsel, above, occ


def _select_body(k, x_hbm, ov_hbm, oi_hbm, row_v, hist_v, cval_v, cidx_v,
                 thr_s, sem):
    wid = lax.axis_index("s") * 2 + lax.axis_index("c")
    lane = lax.iota(jnp.int32, 16)
    for t in range(4):
        r = wid + 32 * t
        @pl.when(r < C)
        def _():
            pltpu.sync_copy(x_hbm.at[r], row_v)
            _hist_pass(row_v, hist_v, lane, 24, jnp.uint32(0), jnp.uint32(0))
            d1, above1, _ = _hist_scan(hist_v, k, jnp.int32(0))
            p1 = lax.convert_element_type(d1, jnp.uint32) << jnp.uint32(24)
            _hist_pass(row_v, hist_v, lane, 16, jnp.uint32(0xFF000000), p1)
            d2, above2, occ2 = _hist_scan(hist_v, k, above1)
            p2 = p1 | (lax.convert_element_type(d2, jnp.uint32) << jnp.uint32(16))
            thr_s[0] = p2
            @pl.when(above2 + occ2 > CAP)
            def _():
                _hist_pass(row_v, hist_v, lane, 8, jnp.uint32(0xFFFF0000), p2)
                d3, _, _ = _hist_scan(hist_v, k, above2)
                thr_s[0] = p2 | (lax.convert_element_type(d3, jnp.uint32)
                                 << jnp.uint32(8))
            thr = thr_s[0]

            def fill(i, _):
                for u in range(4):
                    cval_v[pl.ds(i * 64 + u * 16, 16)] = jnp.full(
                        (16,), NEG, jnp.float32)
                    cidx_v[pl.ds(i * 64 + u * 16, 16)] = jnp.full(
                        (16,), MAXI, jnp.int32)
                return 0
            lax.fori_loop(0, CAP // 64, fill, 0)

            @plsc.parallel_loop(0, _NGROUP, carry=jnp.zeros((16,), jnp.int32))
            def _(i, ptrv):
                base = i * (16 * _UNR)
                datas = []
                for u in range(_UNR):
                    v = row_v[pl.ds(base + u * 16, 16)]
                    m = _key_u32(v) >= thr
                    inc = plsc.cumsum(jnp.where(m, 1, 0))
                    pc = plsc.all_reduce_population_count(m)
                    datas.append((v, m, inc, pc))
                for u in range(_UNR):
                    v, m, inc, pc = datas[u]
                    post = ptrv + inc
                    ok = m & (post <= CAP)
                    plsc.store_scatter(cval_v, [post - 1], v, mask=ok)
                    plsc.store_scatter(cidx_v, [post - 1],
                                       base + u * 16 + lane, mask=ok)
                    ptrv = ptrv + pc
                return ptrv
            pltpu.sync_copy(cval_v, ov_hbm.at[r])
            pltpu.sync_copy(cidx_v, oi_hbm.at[r])


def _select(x, k):
    kern = pl.kernel(
        functools.partial(_select_body, k),
        out_type=(jax.ShapeDtypeStruct((C, CAP), jnp.float32),
                  jax.ShapeDtypeStruct((C, CAP), jnp.int32)),
        mesh=_mesh(),
        scratch_types=[
            pltpu.VMEM((NPAD,), jnp.float32),
            pltpu.VMEM((256 * 16,), jnp.int32),
            pltpu.VMEM((CAP,), jnp.float32),
            pltpu.VMEM((CAP,), jnp.int32),
            pltpu.SMEM((1,), jnp.uint32),
            pltpu.SemaphoreType.DMA,
        ],
        compiler_params=pltpu.CompilerParams(needs_layout_passes=False),
    )
    return kern(x)


def _gather_body(feas_hbm, idx_hbm, out_hbm, idx_v, rows_v, acc_v, sem):
    wid = lax.axis_index("s") * 2 + lax.axis_index("c")
    lane = lax.iota(jnp.int32, 16)
    for t in range(4):
        r = wid + 32 * t
        @pl.when(r < C)
        def _():
            pltpu.sync_copy(idx_hbm.at[r, pl.ds(0, KP1)], idx_v)
            v22 = idx_v[pl.ds(352, 16)]
            idx_v[pl.ds(352, 16)] = jnp.where(352 + lane < K1, v22, 0)
            idx_v[pl.ds(368, 16)] = jnp.zeros((16,), jnp.int32)
            pltpu.async_copy(feas_hbm.at[idx_v], rows_v, sem).wait()

            def body(rr, carry):
                return tuple(
                    carry[j] + rows_v[rr, pl.ds(j * 16, 16)] for j in range(8))
            acc = lax.fori_loop(
                0, K1, body, tuple(jnp.zeros((16,), jnp.float32) for _ in range(8)))
            for j in range(8):
                acc_v[pl.ds(j * 16, 16)] = acc[j]
            pltpu.sync_copy(acc_v, out_hbm.at[r])


def _gather_sums(feas, idxk):
    kern = pl.kernel(
        _gather_body,
        out_type=jax.ShapeDtypeStruct((C, D), jnp.float32),
        mesh=_mesh(),
        scratch_types=[
            pltpu.VMEM((KP1,), jnp.int32),
            pltpu.VMEM((KP1, D), jnp.float32),
            pltpu.VMEM((D,), jnp.float32),
            pltpu.SemaphoreType.DMA,
        ],
        compiler_params=pltpu.CompilerParams(needs_layout_passes=False),
    )
    return kern(feas, idxk)


def kernel(feas_sim, scores, epoch):
    scoresT = _transpose(scores)
    cval1, cidx1 = _select(scoresT, K1)
    sidx1 = _sort_cands(cval1, cidx1)
    idx_max_k = sidx1[:, :K1].T
    sums = _gather_sums(feas_sim, sidx1[:, :KP1])
    centers = sums * (1.0 / 362.0)
    centers = centers / jnp.linalg.norm(centers, axis=1, keepdims=True)
    dis = _dis(centers, feas_sim)
    cval2, cidx2 = _select(dis, K2)
    sidx2 = _sort_cands(cval2, cidx2)
    idx_select = sidx2[:, :K2].reshape(-1)
    idx_select = idx_select + (jnp.asarray(epoch) * 0).astype(jnp.int32)
    labels_select = jnp.repeat(jnp.arange(C, dtype=jnp.int32), K2)
    return (idx_select, labels_select, idx_max_k)
